# baseline (speedup 1.0000x reference)
import functools

import jax
import jax.numpy as jnp
from jax import lax
from jax.experimental import pallas as pl
from jax.experimental.pallas import tpu as pltpu
from jax.experimental.pallas import tpu_sc as plsc

B = 16384
F = 26
D = 16
FIELD = 100000

NW = 32
SPW = B // NW
BLK = 64
NBLK = SPW // BLK
IDX_PER_BLK = BLK * F
ROWS_PER_BLK = IDX_PER_BLK // 128
IDX_ROWS_PER_W = SPW * F // 128


def _sc_body(xi_hbm, xit_hbm, emb_hbm, lin_hbm, cross_hbm, lsum_hbm,
             idx_v, idxt_v, rows_v, lin_v, cross_v, lsum_v, sem):
    c = lax.axis_index("c")
    s = lax.axis_index("s")
    wid = s * 2 + c

    pltpu.sync_copy(xi_hbm.at[pl.ds(wid * IDX_ROWS_PER_W, IDX_ROWS_PER_W)], idx_v)
    pltpu.sync_copy(xit_hbm.at[pl.ds(wid * IDX_ROWS_PER_W, IDX_ROWS_PER_W)], idxt_v)

    def blk_body(blk, carry):
        base_row = blk * ROWS_PER_BLK
        copies = []
        for j in range(ROWS_PER_BLK):
            cp = pltpu.make_async_copy(
                emb_hbm.at[idx_v.at[base_row + j]],
                rows_v.at[pl.ds(j * 128, 128)], sem)
            cp.start()
            copies.append(cp)
            cp2 = pltpu.make_async_copy(
                lin_hbm.at[idxt_v.at[base_row + j]],
                lin_v.at[pl.ds(j * 128, 128)], sem)
            cp2.start()
            copies.append(cp2)
        for cp in copies:
            cp.wait()

        def samp_body(i, carry2):
            r = rows_v[i * F, :]
            s_acc = r
            q_acc = r * r
            for f in range(1, F):
                r = rows_v[i * F + f, :]
                s_acc = s_acc + r
                q_acc = q_acc + r * r
            cross_v[i, :] = 0.5 * (s_acc * s_acc - q_acc)
            return carry2

        lax.fori_loop(0, BLK, samp_body, 0, unroll=False)

        for g in range(BLK // 16):
            acc = lin_v[pl.ds(g * 16, 16)]
            for f in range(1, F):
                acc = acc + lin_v[pl.ds(f * BLK + g * 16, 16)]
            lsum_v[pl.ds(g * 16, 16)] = acc

        out_base = wid * SPW + blk * BLK
        pltpu.sync_copy(cross_v, cross_hbm.at[pl.ds(out_base, BLK)])
        pltpu.sync_copy(lsum_v, lsum_hbm.at[pl.ds(out_base, BLK)])
        return carry

    lax.fori_loop(0, NBLK, blk_body, 0, unroll=False)


_sc_gather = functools.partial(
    pl.kernel,
    mesh=plsc.VectorSubcoreMesh(core_axis_name="c", subcore_axis_name="s"),
    out_type=[
        jax.ShapeDtypeStruct((B, D), jnp.float32),
        jax.ShapeDtypeStruct((B,), jnp.float32),
    ],
    scratch_types=[
        pltpu.VMEM((IDX_ROWS_PER_W, 128), jnp.int32),
        pltpu.VMEM((IDX_ROWS_PER_W, 128), jnp.int32),
        pltpu.VMEM((IDX_PER_BLK, D), jnp.float32),
        pltpu.VMEM((IDX_PER_BLK,), jnp.float32),
        pltpu.VMEM((BLK, D), jnp.float32),
        pltpu.VMEM((BLK,), jnp.float32),
        pltpu.SemaphoreType.DMA,
    ],
    compiler_params=pltpu.CompilerParams(use_tc_tiling_on_sc=False),
)(_sc_body)


def _bn(h, g, b, eps=1e-5):
    m = jnp.mean(h, axis=0, keepdims=True)
    v = jnp.mean((h - m) * (h - m), axis=0, keepdims=True)
    return g * (h - m) * lax.rsqrt(v + eps) + b


def _mlp_body(cross_ref, lsum_ref, g0_ref, b0_ref, w1_ref, b1_ref, g1_ref,
              be1_ref, w2_ref, b2_ref, g2_ref, be2_ref, w3_ref, b3_ref,
              bias_ref, out_ref):
    cross = _bn(cross_ref[...], g0_ref[...], b0_ref[...])
    h = jnp.dot(cross, w1_ref[...], preferred_element_type=jnp.float32)
    h = jnp.maximum(_bn(h + b1_ref[...], g1_ref[...], be1_ref[...]), 0.0)
    h = jnp.dot(h, w2_ref[...], preferred_element_type=jnp.float32)
    h = jnp.maximum(_bn(h + b2_ref[...], g2_ref[...], be2_ref[...]), 0.0)
    mlp = jnp.sum(h * w3_ref[...], axis=1, keepdims=True) + b3_ref[...]
    out_ref[...] = mlp + lsum_ref[...] + bias_ref[...]


def kernel(x, emb_table, lin_table, lin_bias, bn0_gamma, bn0_beta,
           W1, b1, g1, be1, W2, b2, g2, be2, W3, b3):
    offsets = (jnp.arange(F, dtype=x.dtype) * FIELD)[None, :]
    xi = (x + offsets).astype(jnp.int32)
    xi_rows = xi.reshape(B * F // 128, 128)
    xi_t = (xi.reshape(NW, SPW // BLK, BLK, F)
            .transpose(0, 1, 3, 2)
            .reshape(B * F // 128, 128))
    lin_flat = lin_table.reshape(-1)

    cross, lsum = _sc_gather(xi_rows, xi_t, emb_table, lin_flat)

    out = pl.pallas_call(
        _mlp_body,
        out_shape=jax.ShapeDtypeStruct((B, 1), jnp.float32),
    )(
        cross, lsum.reshape(B, 1),
        bn0_gamma.reshape(1, D), bn0_beta.reshape(1, D),
        W1, b1.reshape(1, -1), g1.reshape(1, -1), be1.reshape(1, -1),
        W2, b2.reshape(1, -1), g2.reshape(1, -1), be2.reshape(1, -1),
        W3.reshape(1, -1), b3.reshape(1, 1), lin_bias.reshape(1, 1),
    )
    return out


# speedup vs baseline: 1.0043x; 1.0043x over previous
import functools

import jax
import jax.numpy as jnp
from jax import lax
from jax.experimental import pallas as pl
from jax.experimental.pallas import tpu as pltpu
from jax.experimental.pallas import tpu_sc as plsc

B = 16384
F = 26
D = 16
FIELD = 100000

NW = 32
SPW = B // NW
BLK = 64
NBLK = SPW // BLK
IDX_PER_BLK = BLK * F
ROWS_PER_BLK = IDX_PER_BLK // 128
IDX_ROWS_PER_W = SPW * F // 128


def _sc_body(xi_hbm, xit_hbm, emb_hbm, lin_hbm, cross_hbm, lsum_hbm,
             idx_v, idxt_v, rows_v, lin_v, cross_v, lsum_v, sem):
    c = lax.axis_index("c")
    s = lax.axis_index("s")
    wid = s * 2 + c

    pltpu.sync_copy(xi_hbm.at[pl.ds(wid * IDX_ROWS_PER_W, IDX_ROWS_PER_W)], idx_v)
    pltpu.sync_copy(xit_hbm.at[pl.ds(wid * IDX_ROWS_PER_W, IDX_ROWS_PER_W)], idxt_v)

    def blk_body(blk, carry):
        base_row = blk * ROWS_PER_BLK
        copies = []
        for j in range(ROWS_PER_BLK):
            cp = pltpu.make_async_copy(
                emb_hbm.at[idx_v.at[base_row + j]],
                rows_v.at[pl.ds(j * 128, 128)], sem)
            cp.start()
            copies.append(cp)
            cp2 = pltpu.make_async_copy(
                lin_hbm.at[idxt_v.at[base_row + j]],
                lin_v.at[pl.ds(j * 128, 128)], sem)
            cp2.start()
            copies.append(cp2)
        for cp in copies:
            cp.wait()

        def samp_body(i, carry2):
            r = rows_v[i * F, :]
            s_acc = r
            q_acc = r * r
            for f in range(1, F):
                r = rows_v[i * F + f, :]
                s_acc = s_acc + r
                q_acc = q_acc + r * r
            cross_v[i, :] = 0.5 * (s_acc * s_acc - q_acc)
            return carry2

        lax.fori_loop(0, BLK, samp_body, 0, unroll=False)

        for g in range(BLK // 16):
            acc = lin_v[pl.ds(g * 16, 16)]
            for f in range(1, F):
                acc = acc + lin_v[pl.ds(f * BLK + g * 16, 16)]
            lsum_v[pl.ds(g * 16, 16)] = acc

        out_base = wid * SPW + blk * BLK
        pltpu.sync_copy(cross_v, cross_hbm.at[pl.ds(out_base, BLK)])
        pltpu.sync_copy(lsum_v, lsum_hbm.at[pl.ds(out_base, BLK)])
        return carry

    lax.fori_loop(0, NBLK, blk_body, 0, unroll=False)


_sc_gather = functools.partial(
    pl.kernel,
    mesh=plsc.VectorSubcoreMesh(core_axis_name="c", subcore_axis_name="s"),
    out_type=[
        jax.ShapeDtypeStruct((B, D), jnp.float32),
        jax.ShapeDtypeStruct((B,), jnp.float32),
    ],
    scratch_types=[
        pltpu.VMEM((IDX_ROWS_PER_W, 128), jnp.int32),
        pltpu.VMEM((IDX_ROWS_PER_W, 128), jnp.int32),
        pltpu.VMEM((IDX_PER_BLK, D), jnp.float32),
        pltpu.VMEM((IDX_PER_BLK,), jnp.float32),
        pltpu.VMEM((BLK, D), jnp.float32),
        pltpu.VMEM((BLK,), jnp.float32),
        pltpu.SemaphoreType.DMA,
    ],
    compiler_params=pltpu.CompilerParams(use_tc_tiling_on_sc=False),
)(_sc_body)


def _bn(h, ones_row, g, b, eps=1e-5):
    m = jnp.dot(ones_row, h, preferred_element_type=jnp.float32)
    ms = jnp.dot(ones_row, h * h, preferred_element_type=jnp.float32)
    scale = g * lax.rsqrt(ms - m * m + eps)
    shift = b - m * scale
    return h * scale + shift


def _mlp_body(cross_ref, lsum_ref, g0_ref, b0_ref, w1_ref, b1_ref, g1_ref,
              be1_ref, w2_ref, b2_ref, g2_ref, be2_ref, w3_ref, b3_ref,
              bias_ref, out_ref):
    ones_row = jnp.full((1, B), 1.0 / B, dtype=jnp.float32)
    cross = _bn(cross_ref[...], ones_row, g0_ref[...], b0_ref[...])
    h = jnp.dot(cross, w1_ref[...], preferred_element_type=jnp.float32)
    h = jnp.maximum(_bn(h + b1_ref[...], ones_row, g1_ref[...], be1_ref[...]), 0.0)
    h = jnp.dot(h, w2_ref[...], preferred_element_type=jnp.float32)
    h = jnp.maximum(_bn(h + b2_ref[...], ones_row, g2_ref[...], be2_ref[...]), 0.0)
    mlp = jnp.dot(h, w3_ref[...], preferred_element_type=jnp.float32)
    out_ref[...] = mlp + b3_ref[...] + lsum_ref[...] + bias_ref[...]


def kernel(x, emb_table, lin_table, lin_bias, bn0_gamma, bn0_beta,
           W1, b1, g1, be1, W2, b2, g2, be2, W3, b3):
    offsets = (jnp.arange(F, dtype=x.dtype) * FIELD)[None, :]
    xi = (x + offsets).astype(jnp.int32)
    xi_rows = xi.reshape(B * F // 128, 128)
    xi_t = (xi.reshape(NW, SPW // BLK, BLK, F)
            .transpose(0, 1, 3, 2)
            .reshape(B * F // 128, 128))
    lin_flat = lin_table.reshape(-1)

    cross, lsum = _sc_gather(xi_rows, xi_t, emb_table, lin_flat)

    out = pl.pallas_call(
        _mlp_body,
        out_shape=jax.ShapeDtypeStruct((B, 1), jnp.float32),
    )(
        cross, lsum.reshape(B, 1),
        bn0_gamma.reshape(1, D), bn0_beta.reshape(1, D),
        W1, b1.reshape(1, -1), g1.reshape(1, -1), be1.reshape(1, -1),
        W2, b2.reshape(1, -1), g2.reshape(1, -1), be2.reshape(1, -1),
        W3, b3.reshape(1, 1), lin_bias.reshape(1, 1),
    )
    return out
